# Initial kernel scaffold; baseline (speedup 1.0000x reference)
import jax
import jax.numpy as jnp
from jax.experimental import pallas as pl


def kernel(q, k, v, projection_matrix):
    raise NotImplementedError("write your pallas kernel here")


# speedup vs baseline: 2.5733x; 2.5733x over previous
import functools
import math

import jax
import jax.numpy as jnp
from jax.experimental import pallas as pl
from jax.experimental.pallas import tpu as pltpu

_D_MODEL = 768
_N_BUCKETS = 64


def _attn_kernel(blk, seq, idxs_ref, q_ref, k_ref, v_ref, o_ref, qg, kg, vg, sem):
    b = pl.program_id(0)
    g = pl.program_id(1)
    base = b * seq + g * blk

    def start_row(j, carry):
        idx = idxs_ref[base + j]
        pltpu.make_async_copy(q_ref.at[b, idx], qg.at[j], sem).start()
        pltpu.make_async_copy(k_ref.at[b, idx], kg.at[j], sem).start()
        pltpu.make_async_copy(v_ref.at[b, idx], vg.at[j], sem).start()
        return carry

    jax.lax.fori_loop(0, blk, start_row, 0)

    def wait_row(j, carry):
        idx = idxs_ref[base + j]
        pltpu.make_async_copy(q_ref.at[b, idx], qg.at[j], sem).wait()
        pltpu.make_async_copy(k_ref.at[b, idx], kg.at[j], sem).wait()
        pltpu.make_async_copy(v_ref.at[b, idx], vg.at[j], sem).wait()
        return carry

    jax.lax.fori_loop(0, blk, wait_row, 0)

    qv = qg[...]
    kv = kg[...]
    vv = vg[...]
    s = jax.lax.dot_general(qv, kv, (((1,), (1,)), ((), ())),
                            preferred_element_type=jnp.float32)
    s = s * (1.0 / math.sqrt(_D_MODEL))
    m = jnp.max(s, axis=-1, keepdims=True)
    e = jnp.exp(s - m)
    p = e / jnp.sum(e, axis=-1, keepdims=True)
    o = jax.lax.dot_general(p, vv, (((1,), (0,)), ((), ())),
                            preferred_element_type=jnp.float32)
    o_ref[0] = o * (1.0 / _N_BUCKETS)


def _bucket_ids(q, projection_matrix):
    projected = jnp.matmul(q, projection_matrix)
    hashes = jnp.sign(projected)
    bucket_range = jnp.asarray([2.0 ** i for i in range(_N_BUCKETS // 2)],
                               dtype=jnp.float32)
    ids = jnp.sum(hashes * bucket_range, axis=-1)
    return ids.astype(jnp.int32) % _N_BUCKETS


def kernel(q, k, v, projection_matrix):
    batch, seq, d = q.shape
    blk = seq // _N_BUCKETS

    ids = _bucket_ids(q, projection_matrix)
    idxs = jnp.argsort(ids, axis=-1)
    idxs_flat = idxs.reshape(-1).astype(jnp.int32)

    grid_spec = pltpu.PrefetchScalarGridSpec(
        num_scalar_prefetch=1,
        grid=(batch, _N_BUCKETS),
        in_specs=[
            pl.BlockSpec(memory_space=pl.ANY),
            pl.BlockSpec(memory_space=pl.ANY),
            pl.BlockSpec(memory_space=pl.ANY),
        ],
        out_specs=pl.BlockSpec((1, blk, d), lambda b, g, idxs: (b, g, 0)),
        scratch_shapes=[
            pltpu.VMEM((blk, d), jnp.float32),
            pltpu.VMEM((blk, d), jnp.float32),
            pltpu.VMEM((blk, d), jnp.float32),
            pltpu.SemaphoreType.DMA,
        ],
    )
    return pl.pallas_call(
        functools.partial(_attn_kernel, blk, seq),
        grid_spec=grid_spec,
        out_shape=jax.ShapeDtypeStruct((batch, seq, d), jnp.float32),
    )(idxs_flat, q, k, v)
